# Initial kernel scaffold; baseline (speedup 1.0000x reference)
import functools

import jax
import jax.numpy as jnp
from jax import lax
from jax.experimental import pallas as pl
from jax.experimental.pallas import tpu as pltpu
from jax.experimental.pallas import tpu_sc as plsc

NC = 2
NS = 16
NW = NC * NS
LANES = 16
CHUNK = 128


def _tc_hlin(x, w_pad, b_pad):
    n = x.shape[0]

    def body(x_ref, w_ref, b_ref, o_ref):
        o_ref[...] = (
            jnp.dot(x_ref[...], w_ref[...], preferred_element_type=jnp.float32)
            + b_ref[...]
        )

    return pl.pallas_call(
        body,
        out_shape=jax.ShapeDtypeStruct((n, LANES), jnp.float32),
    )(x, w_pad, b_pad)


def _tc_scale(cnt0, cnt1, hlin):
    n = hlin.shape[0]

    def body(c0_ref, c1_ref, hl_ref, s_ref, dinv_ref):
        deg = c0_ref[...] + c1_ref[...] + 1.0
        dinv = lax.rsqrt(deg)
        dinv_ref[...] = dinv
        s_ref[...] = dinv * hl_ref[...]

    return pl.pallas_call(
        body,
        out_shape=[
            jax.ShapeDtypeStruct((n, LANES), jnp.float32),
            jax.ShapeDtypeStruct((n, LANES), jnp.float32),
        ],
    )(cnt0, cnt1, hlin)


def _tc_out(m0, m1, dinv, hlin, w_out_pad, b_out_pad):
    n = hlin.shape[0]
    c = w_out_pad.shape[1]

    def body(m0_ref, m1_ref, dv_ref, hl_ref, w_ref, b_ref, h_ref, z_ref):
        dinv = dv_ref[...]
        m = m0_ref[...] + m1_ref[...]
        h = jnp.maximum(dinv * m + dinv * dinv * hl_ref[...], 0.0)
        h_ref[...] = h
        z_ref[...] = (
            jnp.dot(h, w_ref[...], preferred_element_type=jnp.float32)
            + b_ref[...]
        )

    return pl.pallas_call(
        body,
        out_shape=[
            jax.ShapeDtypeStruct((n, LANES), jnp.float32),
            jax.ShapeDtypeStruct((n, c), jnp.float32),
        ],
    )(m0, m1, dinv, hlin, w_out_pad, b_out_pad)


def _sc_histogram(cols2d, zeros_rows, ones_blk, n_acc, r_per_w):
    rows_per_sub = n_acc // NS
    mesh = plsc.VectorSubcoreMesh(core_axis_name="c", subcore_axis_name="s")

    @functools.partial(
        pl.kernel,
        out_type=jax.ShapeDtypeStruct((NC, n_acc, LANES), jnp.float32),
        mesh=mesh,
        scratch_types=[
            pltpu.VMEM((r_per_w, CHUNK), jnp.int32),
            pltpu.VMEM((CHUNK, LANES), jnp.float32),
            pltpu.VMEM((rows_per_sub, LANES), jnp.float32),
            pltpu.VMEM_SHARED((n_acc, LANES), jnp.float32),
        ],
    )
    def k(cols_hbm, zeros_hbm, ones_hbm, out_hbm, cols_v, ones_v, zv, acc_sh):
        cid = lax.axis_index("c")
        sid = lax.axis_index("s")
        wid = cid * NS + sid
        sl = pl.ds(sid * rows_per_sub, rows_per_sub)
        pltpu.sync_copy(zeros_hbm, zv)
        pltpu.sync_copy(zv, acc_sh.at[sl])
        pltpu.sync_copy(ones_hbm, ones_v)
        pltpu.sync_copy(cols_hbm.at[wid], cols_v)
        plsc.subcore_barrier()

        @pl.loop(0, r_per_w)
        def _(j):
            pltpu.sync_copy(ones_v, acc_sh.at[cols_v.at[j]], add=True)

        plsc.subcore_barrier()
        pltpu.sync_copy(acc_sh.at[sl], zv)
        pltpu.sync_copy(zv, out_hbm.at[cid].at[sl])

    return k(cols2d, zeros_rows, ones_blk)


def _sc_messages(s_pad, rows2d, cols2d, zeros_rows, n_acc, r_per_w):
    rows_per_sub = n_acc // NS
    mesh = plsc.VectorSubcoreMesh(core_axis_name="c", subcore_axis_name="s")

    @functools.partial(
        pl.kernel,
        out_type=jax.ShapeDtypeStruct((NC, n_acc, LANES), jnp.float32),
        mesh=mesh,
        scratch_types=[
            pltpu.VMEM((r_per_w, CHUNK), jnp.int32),
            pltpu.VMEM((r_per_w, CHUNK), jnp.int32),
            pltpu.VMEM((CHUNK, LANES), jnp.float32),
            pltpu.VMEM((rows_per_sub, LANES), jnp.float32),
            pltpu.VMEM_SHARED((n_acc, LANES), jnp.float32),
        ],
    )
    def k(s_hbm, rows_hbm, cols_hbm, zeros_hbm, out_hbm,
          rows_v, cols_v, msg_v, zv, acc_sh):
        cid = lax.axis_index("c")
        sid = lax.axis_index("s")
        wid = cid * NS + sid
        sl = pl.ds(sid * rows_per_sub, rows_per_sub)
        pltpu.sync_copy(zeros_hbm, zv)
        pltpu.sync_copy(zv, acc_sh.at[sl])
        pltpu.sync_copy(rows_hbm.at[wid], rows_v)
        pltpu.sync_copy(cols_hbm.at[wid], cols_v)
        plsc.subcore_barrier()

        @pl.loop(0, r_per_w)
        def _(j):
            pltpu.sync_copy(s_hbm.at[rows_v.at[j]], msg_v)
            pltpu.sync_copy(msg_v, acc_sh.at[cols_v.at[j]], add=True)

        plsc.subcore_barrier()
        pltpu.sync_copy(acc_sh.at[sl], zv)
        pltpu.sync_copy(zv, out_hbm.at[cid].at[sl])

    return k(s_pad, rows2d, cols2d, zeros_rows)


def kernel(x, edge_index, W_gcn, b_gcn, W_out, b_out):
    n, d = x.shape
    h_dim = W_gcn.shape[1]
    e = edge_index.shape[1]

    r_per_w = -(-e // (NW * CHUNK))
    e_pad = NW * r_per_w * CHUNK
    n_acc = n + LANES

    row = edge_index[0]
    col = edge_index[1]
    pad = e_pad - e
    row_p = jnp.concatenate([row, jnp.zeros((pad,), jnp.int32)])
    col_p = jnp.concatenate([col, jnp.full((pad,), n, jnp.int32)])
    rows2d = row_p.reshape(NW, r_per_w, CHUNK)
    cols2d = col_p.reshape(NW, r_per_w, CHUNK)

    w_pad = jnp.zeros((d, LANES), jnp.float32).at[:, :h_dim].set(W_gcn)
    b_pad = jnp.zeros((1, LANES), jnp.float32).at[0, :h_dim].set(b_gcn)
    w_out_pad = jnp.zeros((LANES, W_out.shape[1]), jnp.float32).at[:h_dim, :].set(W_out)
    b_out_pad = b_out.reshape(1, -1)

    zeros_rows = jnp.zeros((n_acc // NS, LANES), jnp.float32)
    ones_blk = jnp.ones((CHUNK, LANES), jnp.float32)

    hlin = _tc_hlin(x, w_pad, b_pad)
    cnt = _sc_histogram(cols2d, zeros_rows, ones_blk, n_acc, r_per_w)
    s, dinv = _tc_scale(cnt[0, :n], cnt[1, :n], hlin)
    s_pad = jnp.concatenate([s, jnp.zeros((n_acc - n, LANES), jnp.float32)])
    macc = _sc_messages(s_pad, rows2d, cols2d, zeros_rows, n_acc, r_per_w)
    h_full, z = _tc_out(macc[0, :n], macc[1, :n], dinv, hlin, w_out_pad, b_out_pad)
    return (h_full[:, :h_dim], z)


# speedup vs baseline: 45.1153x; 45.1153x over previous
import functools

import jax
import jax.numpy as jnp
from jax import lax
from jax.experimental import pallas as pl
from jax.experimental.pallas import tpu as pltpu
from jax.experimental.pallas import tpu_sc as plsc

NC = 2
NS = 16
NW = NC * NS
CHUNK = 128
H = 3


def _tc_hlin(x, w, b):
    n = x.shape[0]
    h = w.shape[1]

    def body(x_ref, w_ref, b_ref, o_ref):
        o_ref[...] = (
            lax.dot_general(w_ref[...], x_ref[...],
                            (((0,), (1,)), ((), ())),
                            preferred_element_type=jnp.float32)
            + b_ref[...]
        )

    return pl.pallas_call(
        body,
        out_shape=jax.ShapeDtypeStruct((h, n), jnp.float32),
    )(x, w, b)


def _tc_scale(cnt0, cnt1, hlin_t):
    n = hlin_t.shape[1]

    def body(c0_ref, c1_ref, hl_ref, s_ref, dinv_ref):
        deg = c0_ref[...] + c1_ref[...] + 1.0
        dinv = lax.rsqrt(deg)
        dinv_ref[...] = dinv
        s_ref[...] = dinv * hl_ref[...]

    return pl.pallas_call(
        body,
        out_shape=[
            jax.ShapeDtypeStruct((H, n), jnp.float32),
            jax.ShapeDtypeStruct((1, n), jnp.float32),
        ],
    )(cnt0, cnt1, hlin_t)


def _tc_out(ma0, ma1, ma2, dinv, hlin_t, w_out, b_out):
    n = hlin_t.shape[1]
    c = w_out.shape[1]

    def body(ma0_ref, ma1_ref, ma2_ref, dv_ref, hl_ref, w_ref, b_ref,
             h_ref, z_ref):
        dinv = dv_ref[0, :]
        for l, ma in enumerate((ma0_ref, ma1_ref, ma2_ref)):
            m_l = ma[0, :] + ma[1, :]
            h_ref[l, :] = jnp.maximum(
                dinv * m_l + dinv * dinv * hl_ref[l, :], 0.0)
        z_ref[...] = (
            lax.dot_general(w_ref[...], h_ref[...],
                            (((0,), (0,)), ((), ())),
                            preferred_element_type=jnp.float32)
            + b_ref[...]
        )

    return pl.pallas_call(
        body,
        out_shape=[
            jax.ShapeDtypeStruct((H, n), jnp.float32),
            jax.ShapeDtypeStruct((c, n), jnp.float32),
        ],
    )(ma0, ma1, ma2, dinv, hlin_t, w_out, b_out)


def _sc_histogram(cols2d, zeros_rows, ones_blk, n_acc, r_per_w):
    rows_per_sub = n_acc // NS
    mesh = plsc.VectorSubcoreMesh(core_axis_name="c", subcore_axis_name="s")

    @functools.partial(
        pl.kernel,
        out_type=jax.ShapeDtypeStruct((NC, n_acc), jnp.float32),
        mesh=mesh,
        scratch_types=[
            pltpu.VMEM((r_per_w, CHUNK), jnp.int32),
            pltpu.VMEM((CHUNK,), jnp.float32),
            pltpu.VMEM((rows_per_sub,), jnp.float32),
            pltpu.VMEM_SHARED((n_acc,), jnp.float32),
        ],
    )
    def k(cols_hbm, zeros_hbm, ones_hbm, out_hbm, cols_v, ones_v, zv, acc_sh):
        cid = lax.axis_index("c")
        sid = lax.axis_index("s")
        wid = cid * NS + sid
        sl = pl.ds(sid * rows_per_sub, rows_per_sub)
        pltpu.sync_copy(zeros_hbm, zv)
        pltpu.sync_copy(zv, acc_sh.at[sl])
        pltpu.sync_copy(ones_hbm, ones_v)
        pltpu.sync_copy(cols_hbm.at[wid], cols_v)
        plsc.subcore_barrier()

        @pl.loop(0, r_per_w)
        def _(j):
            pltpu.sync_copy(ones_v, acc_sh.at[cols_v.at[j]], add=True)

        plsc.subcore_barrier()
        pltpu.sync_copy(acc_sh.at[sl], zv)
        pltpu.sync_copy(zv, out_hbm.at[cid].at[sl])

    return k(cols2d, zeros_rows, ones_blk)


def _sc_messages(s0, s1, s2, rows2d, cols2d, zeros_rows, n_acc, r_per_w):
    rows_per_sub = n_acc // NS
    mesh = plsc.VectorSubcoreMesh(core_axis_name="c", subcore_axis_name="s")

    @functools.partial(
        pl.kernel,
        out_type=[jax.ShapeDtypeStruct((NC, n_acc), jnp.float32)] * H,
        mesh=mesh,
        scratch_types=(
            [pltpu.VMEM((r_per_w, CHUNK), jnp.int32)] * 2
            + [pltpu.VMEM((CHUNK,), jnp.float32)] * 6
            + [pltpu.VMEM((rows_per_sub,), jnp.float32)]
            + [pltpu.VMEM_SHARED((n_acc,), jnp.float32)] * 3
            + [pltpu.SemaphoreType.DMA] * 2
        ),
    )
    def k(s0_hbm, s1_hbm, s2_hbm, rows_hbm, cols_hbm, zeros_hbm,
          out0_hbm, out1_hbm, out2_hbm,
          rows_v, cols_v, g00, g01, g02, g10, g11, g12, zv,
          acc0, acc1, acc2, sem0, sem1):
        outs = (out0_hbm, out1_hbm, out2_hbm)
        cid = lax.axis_index("c")
        sid = lax.axis_index("s")
        wid = cid * NS + sid
        sl = pl.ds(sid * rows_per_sub, rows_per_sub)
        tabs = (s0_hbm, s1_hbm, s2_hbm)
        accs = (acc0, acc1, acc2)
        bufs = ((g00, g01, g02), (g10, g11, g12))
        sems = (sem0, sem1)

        pltpu.sync_copy(zeros_hbm, zv)
        for a in accs:
            pltpu.sync_copy(zv, a.at[sl])
        pltpu.sync_copy(rows_hbm.at[wid], rows_v)
        pltpu.sync_copy(cols_hbm.at[wid], cols_v)
        plsc.subcore_barrier()

        def start(j, b):
            for l in range(H):
                pltpu.async_copy(tabs[l].at[rows_v.at[j]], bufs[b][l], sems[b])

        def finish(j, b):
            for l in range(H):
                pltpu.make_async_copy(
                    tabs[l].at[rows_v.at[j]], bufs[b][l], sems[b]).wait()

        def scatter(j, b):
            for l in range(H):
                pltpu.sync_copy(bufs[b][l], accs[l].at[cols_v.at[j]], add=True)

        start(0, 0)
        start(1, 1)

        @pl.loop(0, r_per_w, step=2)
        def _(j):
            finish(j, 0)
            scatter(j, 0)

            @pl.when(j + 2 < r_per_w)
            def _():
                start(j + 2, 0)

            finish(j + 1, 1)
            scatter(j + 1, 1)

            @pl.when(j + 3 < r_per_w)
            def _():
                start(j + 3, 1)

        plsc.subcore_barrier()
        for l in range(H):
            pltpu.sync_copy(accs[l].at[sl], zv)
            pltpu.sync_copy(zv, outs[l].at[cid].at[sl])

    return k(s0, s1, s2, rows2d, cols2d, zeros_rows)


def kernel(x, edge_index, W_gcn, b_gcn, W_out, b_out):
    n, d = x.shape
    e = edge_index.shape[1]

    r_per_w = -(-e // (NW * CHUNK))
    r_per_w += r_per_w % 2
    e_pad = NW * r_per_w * CHUNK
    n_acc = (n // (NS * 128) + 1) * (NS * 128)

    row = edge_index[0]
    col = edge_index[1]
    pad = e_pad - e
    row_p = jnp.concatenate([row, jnp.zeros((pad,), jnp.int32)])
    col_p = jnp.concatenate([col, jnp.full((pad,), n, jnp.int32)])
    rows2d = row_p.reshape(NW, r_per_w, CHUNK)
    cols2d = col_p.reshape(NW, r_per_w, CHUNK)

    zeros_rows = jnp.zeros((n_acc // NS,), jnp.float32)
    ones_blk = jnp.ones((CHUNK,), jnp.float32)

    hlin_t = _tc_hlin(x, W_gcn, b_gcn.reshape(H, 1))
    cnt = _sc_histogram(cols2d, zeros_rows, ones_blk, n_acc, r_per_w)
    s_t, dinv = _tc_scale(cnt[0, :n].reshape(1, n), cnt[1, :n].reshape(1, n),
                          hlin_t)
    ma0, ma1, ma2 = _sc_messages(s_t[0], s_t[1], s_t[2], rows2d, cols2d,
                                 zeros_rows, n_acc, r_per_w)
    h_t, z_t = _tc_out(ma0[:, :n], ma1[:, :n], ma2[:, :n], dinv, hlin_t,
                       W_out, b_out.reshape(-1, 1))
    return (h_t.T, z_t.T)


# speedup vs baseline: 70.1297x; 1.5545x over previous
import functools

import jax
import jax.numpy as jnp
from jax import lax
from jax.experimental import pallas as pl
from jax.experimental.pallas import tpu as pltpu
from jax.experimental.pallas import tpu_sc as plsc

NC = 2
NS = 16
NW = NC * NS
CHUNK = 128
H = 3


def _tc_hlin(x, w, b):
    n = x.shape[0]
    h = w.shape[1]

    def body(x_ref, w_ref, b_ref, o_ref):
        o_ref[...] = (
            lax.dot_general(w_ref[...], x_ref[...],
                            (((0,), (1,)), ((), ())),
                            preferred_element_type=jnp.float32)
            + b_ref[...]
        )

    return pl.pallas_call(
        body,
        out_shape=jax.ShapeDtypeStruct((h, n), jnp.float32),
    )(x, w, b)


def _tc_scale(cnt0, cnt1, hlin_t):
    n = hlin_t.shape[1]

    def body(c0_ref, c1_ref, hl_ref, s_ref, dinv_ref):
        deg = c0_ref[...] + c1_ref[...] + 1.0
        dinv = lax.rsqrt(deg)
        dinv_ref[...] = dinv
        s_ref[...] = dinv * hl_ref[...]

    return pl.pallas_call(
        body,
        out_shape=[
            jax.ShapeDtypeStruct((H, n), jnp.float32),
            jax.ShapeDtypeStruct((1, n), jnp.float32),
        ],
    )(cnt0, cnt1, hlin_t)


def _tc_out(ma0, ma1, ma2, dinv, hlin_t, w_out, b_out):
    n = hlin_t.shape[1]
    c = w_out.shape[1]

    def body(ma0_ref, ma1_ref, ma2_ref, dv_ref, hl_ref, w_ref, b_ref,
             h_ref, z_ref):
        dinv = dv_ref[0, :]
        for l, ma in enumerate((ma0_ref, ma1_ref, ma2_ref)):
            m_l = ma[0, :] + ma[1, :]
            h_ref[l, :] = jnp.maximum(
                dinv * m_l + dinv * dinv * hl_ref[l, :], 0.0)
        z_ref[...] = (
            lax.dot_general(w_ref[...], h_ref[...],
                            (((0,), (0,)), ((), ())),
                            preferred_element_type=jnp.float32)
            + b_ref[...]
        )

    return pl.pallas_call(
        body,
        out_shape=[
            jax.ShapeDtypeStruct((H, n), jnp.float32),
            jax.ShapeDtypeStruct((c, n), jnp.float32),
        ],
    )(ma0, ma1, ma2, dinv, hlin_t, w_out, b_out)


def _sc_histogram(cols2d, zeros_rows, ones_blk, n_acc, r_per_w):
    rows_per_sub = n_acc // NS
    mesh = plsc.VectorSubcoreMesh(core_axis_name="c", subcore_axis_name="s")

    @functools.partial(
        pl.kernel,
        out_type=jax.ShapeDtypeStruct((NC, n_acc), jnp.float32),
        mesh=mesh,
        scratch_types=[
            pltpu.VMEM((r_per_w, CHUNK), jnp.int32),
            pltpu.VMEM((CHUNK,), jnp.float32),
            pltpu.VMEM((rows_per_sub,), jnp.float32),
            pltpu.VMEM_SHARED((n_acc,), jnp.float32),
            pltpu.SemaphoreType.DMA,
        ],
    )
    def k(cols_hbm, zeros_hbm, ones_hbm, out_hbm, cols_v, ones_v, zv, acc_sh,
          hsem):
        cid = lax.axis_index("c")
        sid = lax.axis_index("s")
        wid = cid * NS + sid
        sl = pl.ds(sid * rows_per_sub, rows_per_sub)
        pltpu.sync_copy(zeros_hbm, zv)
        pltpu.sync_copy(zv, acc_sh.at[sl])
        pltpu.sync_copy(ones_hbm, ones_v)
        pltpu.sync_copy(cols_hbm.at[wid], cols_v)
        plsc.subcore_barrier()

        @pl.loop(0, r_per_w, step=8)
        def _(j):
            for b in range(8):
                pltpu.async_copy(ones_v, acc_sh.at[cols_v.at[j + b]], hsem,
                                 add=True)
            for b in range(8):
                pltpu.make_async_copy(
                    ones_v, acc_sh.at[cols_v.at[j + b]], hsem).wait()

        plsc.subcore_barrier()
        pltpu.sync_copy(acc_sh.at[sl], zv)
        pltpu.sync_copy(zv, out_hbm.at[cid].at[sl])

    return k(cols2d, zeros_rows, ones_blk)


def _sc_messages(s0, s1, s2, rows2d, cols2d, zeros_rows, n_acc, r_per_w):
    rows_per_sub = n_acc // NS
    mesh = plsc.VectorSubcoreMesh(core_axis_name="c", subcore_axis_name="s")

    @functools.partial(
        pl.kernel,
        out_type=[jax.ShapeDtypeStruct((NC, n_acc), jnp.float32)] * H,
        mesh=mesh,
        scratch_types=(
            [pltpu.VMEM((r_per_w, CHUNK), jnp.int32)] * 2
            + [pltpu.VMEM((CHUNK,), jnp.float32)] * 6
            + [pltpu.VMEM((rows_per_sub,), jnp.float32)]
            + [pltpu.VMEM_SHARED((n_acc,), jnp.float32)] * 3
            + [pltpu.VMEM_SHARED((n_acc,), jnp.float32)] * 3
            + [pltpu.SemaphoreType.DMA] * 4
        ),
    )
    def k(s0_hbm, s1_hbm, s2_hbm, rows_hbm, cols_hbm, zeros_hbm,
          out0_hbm, out1_hbm, out2_hbm,
          rows_v, cols_v, g00, g01, g02, g10, g11, g12, zv,
          tab0, tab1, tab2, acc0, acc1, acc2, gsem0, gsem1, ssem0, ssem1):
        outs = (out0_hbm, out1_hbm, out2_hbm)
        cid = lax.axis_index("c")
        sid = lax.axis_index("s")
        wid = cid * NS + sid
        sl = pl.ds(sid * rows_per_sub, rows_per_sub)
        s_hbm = (s0_hbm, s1_hbm, s2_hbm)
        tabs = (tab0, tab1, tab2)
        accs = (acc0, acc1, acc2)
        bufs = ((g00, g01, g02), (g10, g11, g12))
        gsems = (gsem0, gsem1)
        ssems = (ssem0, ssem1)

        for l in range(H):
            pltpu.sync_copy(s_hbm[l].at[sl], zv)
            pltpu.sync_copy(zv, tabs[l].at[sl])
        pltpu.sync_copy(zeros_hbm, zv)
        for a in accs:
            pltpu.sync_copy(zv, a.at[sl])
        pltpu.sync_copy(rows_hbm.at[wid], rows_v)
        pltpu.sync_copy(cols_hbm.at[wid], cols_v)
        plsc.subcore_barrier()

        def start_g(j, b):
            for l in range(H):
                pltpu.async_copy(tabs[l].at[rows_v.at[j]], bufs[b][l],
                                 gsems[b])

        def wait_g(j, b):
            for l in range(H):
                pltpu.make_async_copy(
                    tabs[l].at[rows_v.at[j]], bufs[b][l], gsems[b]).wait()

        def scatter(j, b):
            for l in range(H):
                pltpu.sync_copy(bufs[b][l], accs[l].at[cols_v.at[j]],
                                add=True)

        start_g(0, 0)
        start_g(1, 1)

        @pl.loop(0, r_per_w, step=2)
        def _(j):
            wait_g(j, 0)
            scatter(j, 0)

            @pl.when(j + 2 < r_per_w)
            def _():
                start_g(j + 2, 0)

            wait_g(j + 1, 1)
            scatter(j + 1, 1)

            @pl.when(j + 3 < r_per_w)
            def _():
                start_g(j + 3, 1)

        plsc.subcore_barrier()
        for l in range(H):
            pltpu.sync_copy(accs[l].at[sl], zv)
            pltpu.sync_copy(zv, outs[l].at[cid].at[sl])

    return k(s0, s1, s2, rows2d, cols2d, zeros_rows)


def kernel(x, edge_index, W_gcn, b_gcn, W_out, b_out):
    n, d = x.shape
    e = edge_index.shape[1]

    r_per_w = -(-e // (NW * CHUNK))
    r_per_w = -(-r_per_w // 8) * 8
    e_pad = NW * r_per_w * CHUNK
    n_acc = (n // (NS * 128) + 1) * (NS * 128)

    row = edge_index[0]
    col = edge_index[1]
    pad = e_pad - e
    row_p = jnp.concatenate([row, jnp.zeros((pad,), jnp.int32)])
    col_p = jnp.concatenate([col, jnp.full((pad,), n, jnp.int32)])
    rows2d = row_p.reshape(NW, r_per_w, CHUNK)
    cols2d = col_p.reshape(NW, r_per_w, CHUNK)

    zeros_rows = jnp.zeros((n_acc // NS,), jnp.float32)
    ones_blk = jnp.ones((CHUNK,), jnp.float32)

    hlin_t = _tc_hlin(x, W_gcn, b_gcn.reshape(H, 1))
    cnt = _sc_histogram(cols2d, zeros_rows, ones_blk, n_acc, r_per_w)
    s_t, dinv = _tc_scale(cnt[0, :n].reshape(1, n), cnt[1, :n].reshape(1, n),
                          hlin_t)
    s_t_pad = jnp.pad(s_t, ((0, 0), (0, n_acc - n)))
    ma0, ma1, ma2 = _sc_messages(s_t_pad[0], s_t_pad[1], s_t_pad[2],
                                 rows2d, cols2d, zeros_rows, n_acc, r_per_w)
    h_t, z_t = _tc_out(ma0[:, :n], ma1[:, :n], ma2[:, :n], dinv, hlin_t,
                       W_out, b_out.reshape(-1, 1))
    return (h_t.T, z_t.T)


# speedup vs baseline: 85.2584x; 1.2157x over previous
import functools

import jax
import jax.numpy as jnp
from jax import lax
from jax.experimental import pallas as pl
from jax.experimental.pallas import tpu as pltpu
from jax.experimental.pallas import tpu_sc as plsc

NC = 2
NS = 16
NW = NC * NS
CHUNK = 128
H = 3


def _tc_hlin(x, w, b, n_acc):
    n = x.shape[0]

    def body(x_ref, w_ref, b_ref, o_ref):
        res = lax.dot_general(w_ref[...], x_ref[...],
                              (((0,), (1,)), ((), ())),
                              preferred_element_type=jnp.float32) + b_ref[...]
        o_ref[...] = jnp.pad(res, ((0, 0), (0, n_acc - n)))

    return pl.pallas_call(
        body,
        out_shape=jax.ShapeDtypeStruct((H, n_acc), jnp.float32),
    )(x, w, b)


def _tc_scale(cnt, hlin_t):
    n_acc = hlin_t.shape[1]

    def body(c_ref, hl_ref, s_ref, dinv_ref):
        deg = c_ref[0, :] + c_ref[1, :] + 1.0
        dinv = lax.rsqrt(deg)
        dinv_ref[0, :] = dinv
        s_ref[...] = dinv[None, :] * hl_ref[...]

    return pl.pallas_call(
        body,
        out_shape=[
            jax.ShapeDtypeStruct((H, n_acc), jnp.float32),
            jax.ShapeDtypeStruct((1, n_acc), jnp.float32),
        ],
    )(cnt, hlin_t)


def _tc_out(ma0, ma1, ma2, dinv, hlin_t, w_out, b_out):
    n_acc = hlin_t.shape[1]
    c = w_out.shape[1]

    def body(ma0_ref, ma1_ref, ma2_ref, dv_ref, hl_ref, w_ref, b_ref,
             h_ref, z_ref):
        dinv = dv_ref[0, :]
        for l, ma in enumerate((ma0_ref, ma1_ref, ma2_ref)):
            m_l = ma[0, :] + ma[1, :]
            h_ref[l, :] = jnp.maximum(
                dinv * m_l + dinv * dinv * hl_ref[l, :], 0.0)
        z_ref[...] = (
            lax.dot_general(w_ref[...], h_ref[...],
                            (((0,), (0,)), ((), ())),
                            preferred_element_type=jnp.float32)
            + b_ref[...]
        )

    return pl.pallas_call(
        body,
        out_shape=[
            jax.ShapeDtypeStruct((H, n_acc), jnp.float32),
            jax.ShapeDtypeStruct((c, n_acc), jnp.float32),
        ],
    )(ma0, ma1, ma2, dinv, hlin_t, w_out, b_out)


def _worker_span(nchunks, r8):
    cid = lax.axis_index("c")
    sid = lax.axis_index("s")
    wid = cid * NS + sid
    start = pl.multiple_of(wid * r8, 8)
    count = jnp.clip(nchunks - wid * r8, 0, r8)
    return cid, sid, start, count


def _sc_histogram(cols2d, zeros_rows, ones_blk, n_acc, nchunks, r8):
    rows_per_sub = n_acc // NS
    mesh = plsc.VectorSubcoreMesh(core_axis_name="c", subcore_axis_name="s")

    @functools.partial(
        pl.kernel,
        out_type=jax.ShapeDtypeStruct((NC, n_acc), jnp.float32),
        mesh=mesh,
        scratch_types=[
            pltpu.VMEM((r8, CHUNK), jnp.int32),
            pltpu.VMEM((CHUNK,), jnp.float32),
            pltpu.VMEM((rows_per_sub,), jnp.float32),
            pltpu.VMEM_SHARED((n_acc,), jnp.float32),
            pltpu.SemaphoreType.DMA,
        ],
    )
    def k(cols_hbm, zeros_hbm, ones_hbm, out_hbm, cols_v, ones_v, zv, acc_sh,
          hsem):
        cid, sid, start, count = _worker_span(nchunks, r8)
        sl = pl.ds(sid * rows_per_sub, rows_per_sub)
        pltpu.sync_copy(zeros_hbm, zv)
        pltpu.sync_copy(zv, acc_sh.at[sl])
        pltpu.sync_copy(ones_hbm, ones_v)
        pltpu.sync_copy(cols_hbm.at[pl.ds(start, r8)], cols_v)
        plsc.subcore_barrier()

        @pl.loop(0, r8, step=8)
        def _(j):
            for b in range(8):
                @pl.when(j + b < count)
                def _():
                    pltpu.async_copy(ones_v, acc_sh.at[cols_v.at[j + b]],
                                     hsem, add=True)
            for b in range(8):
                @pl.when(j + b < count)
                def _():
                    pltpu.make_async_copy(
                        ones_v, acc_sh.at[cols_v.at[j + b]], hsem).wait()

        plsc.subcore_barrier()
        pltpu.sync_copy(acc_sh.at[sl], zv)
        pltpu.sync_copy(zv, out_hbm.at[cid].at[sl])

    return k(cols2d, zeros_rows, ones_blk)


def _sc_messages(s0, s1, s2, rows2d, cols2d, zeros_rows, n_acc, nchunks, r8):
    rows_per_sub = n_acc // NS
    mesh = plsc.VectorSubcoreMesh(core_axis_name="c", subcore_axis_name="s")

    @functools.partial(
        pl.kernel,
        out_type=[jax.ShapeDtypeStruct((NC, n_acc), jnp.float32)] * H,
        mesh=mesh,
        scratch_types=(
            [pltpu.VMEM((r8, CHUNK), jnp.int32)] * 2
            + [pltpu.VMEM((CHUNK,), jnp.float32)] * 6
            + [pltpu.VMEM((rows_per_sub,), jnp.float32)]
            + [pltpu.VMEM_SHARED((n_acc,), jnp.float32)] * 3
            + [pltpu.VMEM_SHARED((n_acc,), jnp.float32)] * 3
            + [pltpu.SemaphoreType.DMA] * 4
        ),
    )
    def k(s0_hbm, s1_hbm, s2_hbm, rows_hbm, cols_hbm, zeros_hbm,
          out0_hbm, out1_hbm, out2_hbm,
          rows_v, cols_v, g00, g01, g02, g10, g11, g12, zv,
          tab0, tab1, tab2, acc0, acc1, acc2, gsem0, gsem1, ssem0, ssem1):
        outs = (out0_hbm, out1_hbm, out2_hbm)
        cid, sid, start, count = _worker_span(nchunks, r8)
        sl = pl.ds(sid * rows_per_sub, rows_per_sub)
        s_hbm = (s0_hbm, s1_hbm, s2_hbm)
        tabs = (tab0, tab1, tab2)
        accs = (acc0, acc1, acc2)
        bufs = ((g00, g01, g02), (g10, g11, g12))
        gsems = (gsem0, gsem1)
        ssems = (ssem0, ssem1)

        for l in range(H):
            pltpu.sync_copy(s_hbm[l].at[sl], zv)
            pltpu.sync_copy(zv, tabs[l].at[sl])
        pltpu.sync_copy(zeros_hbm, zv)
        for a in accs:
            pltpu.sync_copy(zv, a.at[sl])
        pltpu.sync_copy(rows_hbm.at[pl.ds(start, r8)], rows_v)
        pltpu.sync_copy(cols_hbm.at[pl.ds(start, r8)], cols_v)
        plsc.subcore_barrier()

        def start_g(j, b):
            for l in range(H):
                pltpu.async_copy(tabs[l].at[rows_v.at[j]], bufs[b][l],
                                 gsems[b])

        def wait_g(j, b):
            for l in range(H):
                pltpu.make_async_copy(
                    tabs[l].at[rows_v.at[j]], bufs[b][l], gsems[b]).wait()

        def start_s(j, b):
            for l in range(H):
                pltpu.async_copy(bufs[b][l], accs[l].at[cols_v.at[j]],
                                 ssems[b], add=True)

        def wait_s(j, b):
            for l in range(H):
                pltpu.make_async_copy(
                    bufs[b][l], accs[l].at[cols_v.at[j]], ssems[b]).wait()

        @pl.when(0 < count)
        def _():
            start_g(0, 0)

        @pl.when(1 < count)
        def _():
            start_g(1, 1)

        @pl.loop(0, r8, step=2)
        def _(j):
            @pl.when(j < count)
            def _():
                wait_g(j, 0)
                start_s(j, 0)

            @pl.when(j + 1 < count)
            def _():
                wait_g(j + 1, 1)
                start_s(j + 1, 1)

            @pl.when(j + 2 < count)
            def _():
                wait_s(j, 0)
                start_g(j + 2, 0)

            @pl.when(j + 3 < count)
            def _():
                wait_s(j + 1, 1)
                start_g(j + 3, 1)

        for d in (2, 1):
            jt = count - d
            for b in range(2):
                @pl.when(jnp.logical_and(jt >= 0, jt % 2 == b))
                def _(jt=jt, b=b):
                    wait_s(jt, b)

        plsc.subcore_barrier()
        for l in range(H):
            pltpu.sync_copy(accs[l].at[sl], zv)
            pltpu.sync_copy(zv, outs[l].at[cid].at[sl])

    return k(s0, s1, s2, rows2d, cols2d, zeros_rows)


def kernel(x, edge_index, W_gcn, b_gcn, W_out, b_out):
    n, d = x.shape
    e = edge_index.shape[1]

    n_acc = (n // (NS * 128) + 1) * (NS * 128)

    row = edge_index[0]
    col = edge_index[1]
    if e % CHUNK:
        pad = CHUNK - e % CHUNK
        row = jnp.concatenate([row, jnp.zeros((pad,), jnp.int32)])
        col = jnp.concatenate([col, jnp.full((pad,), n, jnp.int32)])
    nchunks = row.shape[0] // CHUNK
    chunks_per_w = -(-nchunks // NW)
    r8 = -(-chunks_per_w // 8) * 8
    pad_rows = NW * r8 - nchunks
    rows2d = jnp.pad(row.reshape(nchunks, CHUNK), ((0, pad_rows), (0, 0)))
    cols2d = jnp.pad(col.reshape(nchunks, CHUNK), ((0, pad_rows), (0, 0)))

    zeros_rows = jnp.zeros((n_acc // NS,), jnp.float32)
    ones_blk = jnp.ones((CHUNK,), jnp.float32)

    hlin_t = _tc_hlin(x, W_gcn, b_gcn.reshape(H, 1), n_acc)
    cnt = _sc_histogram(cols2d, zeros_rows, ones_blk, n_acc, nchunks, r8)
    s_t, dinv = _tc_scale(cnt, hlin_t)
    ma0, ma1, ma2 = _sc_messages(s_t[0], s_t[1], s_t[2], rows2d, cols2d,
                                 zeros_rows, n_acc, nchunks, r8)
    h_t, z_t = _tc_out(ma0, ma1, ma2, dinv, hlin_t,
                       W_out, b_out.reshape(-1, 1))
    return (h_t[:, :n].T, z_t[:, :n].T)
